# baseline (device time: 40055 ns/iter reference)
import jax
import jax.numpy as jnp
from jax import lax
from jax.experimental import pallas as pl
from jax.experimental.pallas import tpu as pltpu

N_DEV = 8


def kernel(x, w_mat, scale_x, scale_w):
    m_per, k = x.shape
    n = w_mat.shape[1]
    n_per = n // N_DEV
    m_tot = m_per * N_DEV

    def body(x_hbm, w_hbm, sx_ref, sw_ref, out_ref,
             xbuf, x8_ref, wbuf, comm_ref, rbuf,
             xload_sem, load_sems, send_sems, recv_sems):
        my = lax.axis_index("i")
        scale = sx_ref[0] * sw_ref[0]

        barrier_sem = pltpu.get_barrier_semaphore()
        for e in range(1, N_DEV):
            pl.semaphore_signal(
                barrier_sem, inc=1,
                device_id=(lax.rem(my + e, N_DEV),),
                device_id_type=pl.DeviceIdType.MESH,
            )

        def w_dma(slot, tgt):
            return pltpu.make_async_copy(
                w_hbm.at[:, pl.ds(tgt * n_per, n_per)],
                wbuf.at[slot],
                load_sems.at[slot],
            )

        x_dma = pltpu.make_async_copy(x_hbm, xbuf, xload_sem)
        x_dma.start()
        dmas = []
        for d in range(1, N_DEV + 1):
            dma = w_dma(d - 1, lax.rem(my + d, N_DEV))
            dma.start()
            dmas.append(dma)
        x_dma.wait()
        x8_ref[...] = xbuf[...].astype(jnp.float8_e4m3fn)

        def wait_recv_block(d):
            src = lax.rem(my - d + N_DEV, N_DEV)
            recv = pltpu.make_async_remote_copy(
                src_ref=comm_ref.at[0],
                dst_ref=rbuf.at[pl.ds(src * m_per, m_per), :],
                send_sem=send_sems.at[0],
                recv_sem=recv_sems.at[d],
                device_id=(src,),
                device_id_type=pl.DeviceIdType.MESH,
            )
            recv.wait_recv()
            out_ref[pl.ds(src * m_per, m_per), :] = (
                rbuf[pl.ds(src * m_per, m_per), :].astype(jnp.float32)
            )

        send_rdmas = []
        for d in range(1, N_DEV + 1):
            slot = d - 1
            tgt = lax.rem(my + d, N_DEV)
            dmas[d - 1].wait()

            w8 = wbuf[slot].astype(jnp.float8_e5m2)
            chunk = lax.dot_general(
                x8_ref[...], w8, (((1,), (0,)), ((), ())),
                preferred_element_type=jnp.float32,
            ) * scale

            if d == N_DEV:
                out_ref[pl.ds(my * m_per, m_per), :] = chunk
            else:
                comm_ref[d - 1, :, :] = chunk.astype(jnp.bfloat16)
                if d == 1:
                    pl.semaphore_wait(barrier_sem, N_DEV - 1)
                rdma = pltpu.make_async_remote_copy(
                    src_ref=comm_ref.at[d - 1],
                    dst_ref=rbuf.at[pl.ds(my * m_per, m_per), :],
                    send_sem=send_sems.at[d],
                    recv_sem=recv_sems.at[d],
                    device_id=(tgt,),
                    device_id_type=pl.DeviceIdType.MESH,
                )
                rdma.start()
                send_rdmas.append(rdma)

        for d in range(1, N_DEV):
            wait_recv_block(d)

        for rdma in send_rdmas:
            rdma.wait_send()

    return pl.pallas_call(
        body,
        out_shape=jax.ShapeDtypeStruct((m_tot, n_per), jnp.float32),
        in_specs=[
            pl.BlockSpec(memory_space=pl.ANY),
            pl.BlockSpec(memory_space=pl.ANY),
            pl.BlockSpec(memory_space=pltpu.VMEM),
            pl.BlockSpec(memory_space=pltpu.VMEM),
        ],
        out_specs=pl.BlockSpec(memory_space=pltpu.VMEM),
        scratch_shapes=[
            pltpu.VMEM((m_per, k), jnp.float32),
            pltpu.VMEM((m_per, k), jnp.float8_e4m3fn),
            pltpu.VMEM((N_DEV, k, n_per), jnp.float32),
            pltpu.VMEM((N_DEV - 1, m_per, n_per), jnp.bfloat16),
            pltpu.VMEM((m_tot, n_per), jnp.bfloat16),
            pltpu.SemaphoreType.DMA,
            pltpu.SemaphoreType.DMA((N_DEV,)),
            pltpu.SemaphoreType.DMA((N_DEV,)),
            pltpu.SemaphoreType.DMA((N_DEV,)),
        ],
        compiler_params=pltpu.CompilerParams(
            vmem_limit_bytes=120 * 1024 * 1024,
            collective_id=0,
        ),
    )(x, w_mat, scale_x, scale_w)


# device time: 36913 ns/iter; 1.0851x vs baseline; 1.0851x over previous
import jax
import jax.numpy as jnp
from jax import lax
from jax.experimental import pallas as pl
from jax.experimental.pallas import tpu as pltpu

N_DEV = 8


def kernel(x, w_mat, scale_x, scale_w):
    m_per, k = x.shape
    n = w_mat.shape[1]
    n_per = n // N_DEV
    m_tot = m_per * N_DEV

    def body(x8_ref, w_hbm, sx_ref, sw_ref, out_ref,
             wbuf, comm_ref, rbuf,
             load_sems, send_sems, recv_sems):
        my = lax.axis_index("i")
        scale = sx_ref[0] * sw_ref[0]

        barrier_sem = pltpu.get_barrier_semaphore()
        for e in range(1, N_DEV):
            pl.semaphore_signal(
                barrier_sem, inc=1,
                device_id=(lax.rem(my + e, N_DEV),),
                device_id_type=pl.DeviceIdType.MESH,
            )

        def w_dma(slot, tgt):
            return pltpu.make_async_copy(
                w_hbm.at[:, pl.ds(tgt * n_per, n_per)],
                wbuf.at[slot],
                load_sems.at[slot],
            )

        dmas = [w_dma(0, lax.rem(my + 1, N_DEV)),
                w_dma(1, lax.rem(my + 2, N_DEV))]
        dmas[0].start()
        dmas[1].start()

        def wait_recv_block(d):
            src = lax.rem(my - d + N_DEV, N_DEV)
            recv = pltpu.make_async_remote_copy(
                src_ref=comm_ref.at[0],
                dst_ref=rbuf.at[pl.ds(src * m_per, m_per), :],
                send_sem=send_sems.at[0],
                recv_sem=recv_sems.at[d],
                device_id=(src,),
                device_id_type=pl.DeviceIdType.MESH,
            )
            recv.wait_recv()
            out_ref[pl.ds(src * m_per, m_per), :] = (
                rbuf[pl.ds(src * m_per, m_per), :].astype(jnp.float32)
            )

        send_rdmas = []
        for d in range(1, N_DEV + 1):
            slot = (d - 1) % 3
            tgt = lax.rem(my + d, N_DEV)
            if d + 2 <= N_DEV:
                nxt = w_dma((d + 1) % 3, lax.rem(my + d + 2, N_DEV))
                nxt.start()
                dmas.append(nxt)
            dmas[d - 1].wait()

            w8 = wbuf[slot].astype(jnp.float8_e5m2)
            chunk = lax.dot_general(
                x8_ref[...], w8, (((1,), (0,)), ((), ())),
                preferred_element_type=jnp.float32,
            ) * scale

            if d == N_DEV:
                out_ref[pl.ds(my * m_per, m_per), :] = chunk
            else:
                comm_ref[d - 1, :, :] = chunk.astype(jnp.bfloat16)
                if d == 1:
                    pl.semaphore_wait(barrier_sem, N_DEV - 1)
                rdma = pltpu.make_async_remote_copy(
                    src_ref=comm_ref.at[d - 1],
                    dst_ref=rbuf.at[pl.ds(my * m_per, m_per), :],
                    send_sem=send_sems.at[d],
                    recv_sem=recv_sems.at[d],
                    device_id=(tgt,),
                    device_id_type=pl.DeviceIdType.MESH,
                )
                rdma.start()
                send_rdmas.append(rdma)

        for d in range(1, N_DEV):
            wait_recv_block(d)

        for rdma in send_rdmas:
            rdma.wait_send()

    return pl.pallas_call(
        body,
        out_shape=jax.ShapeDtypeStruct((m_tot, n_per), jnp.float32),
        in_specs=[
            pl.BlockSpec(memory_space=pltpu.VMEM),
            pl.BlockSpec(memory_space=pl.ANY),
            pl.BlockSpec(memory_space=pltpu.VMEM),
            pl.BlockSpec(memory_space=pltpu.VMEM),
        ],
        out_specs=pl.BlockSpec(memory_space=pltpu.VMEM),
        scratch_shapes=[
            pltpu.VMEM((3, k, n_per), jnp.float32),
            pltpu.VMEM((N_DEV - 1, m_per, n_per), jnp.bfloat16),
            pltpu.VMEM((m_tot, n_per), jnp.bfloat16),
            pltpu.SemaphoreType.DMA((3,)),
            pltpu.SemaphoreType.DMA((N_DEV,)),
            pltpu.SemaphoreType.DMA((N_DEV,)),
        ],
        compiler_params=pltpu.CompilerParams(
            vmem_limit_bytes=120 * 1024 * 1024,
            collective_id=0,
        ),
    )(x.astype(jnp.float8_e4m3fn), w_mat, scale_x, scale_w)


# device time: 32872 ns/iter; 1.2185x vs baseline; 1.1229x over previous
import jax
import jax.numpy as jnp
from jax import lax
from jax.experimental import pallas as pl
from jax.experimental.pallas import tpu as pltpu

N_DEV = 8


def kernel(x, w_mat, scale_x, scale_w):
    m_per, k = x.shape
    n = w_mat.shape[1]
    n_per = n // N_DEV
    m_tot = m_per * N_DEV

    def body(x_hbm, w_hbm, sx_ref, sw_ref, out_ref,
             xbuf, x8_ref, wbuf, comm_ref, rbuf,
             xload_sem, load_sems, send_sems, recv_sems):
        my = lax.axis_index("i")
        scale = sx_ref[0] * sw_ref[0]

        barrier_sem = pltpu.get_barrier_semaphore()
        for e in range(1, N_DEV):
            pl.semaphore_signal(
                barrier_sem, inc=1,
                device_id=(lax.rem(my + e, N_DEV),),
                device_id_type=pl.DeviceIdType.MESH,
            )

        def w_dma(slot, tgt):
            return pltpu.make_async_copy(
                w_hbm.at[:, pl.ds(tgt * n_per, n_per)],
                wbuf.at[slot],
                load_sems.at[slot],
            )

        x_dma = pltpu.make_async_copy(x_hbm, xbuf, xload_sem)
        x_dma.start()
        dmas = [w_dma(0, lax.rem(my + 1, N_DEV)),
                w_dma(1, lax.rem(my + 2, N_DEV))]
        dmas[0].start()
        dmas[1].start()
        x_dma.wait()
        x8_ref[...] = xbuf[...].astype(jnp.float8_e4m3fn)

        def wait_recv_block(d):
            src = lax.rem(my - d + N_DEV, N_DEV)
            recv = pltpu.make_async_remote_copy(
                src_ref=comm_ref.at[0],
                dst_ref=rbuf.at[pl.ds(src * m_per, m_per), :],
                send_sem=send_sems.at[0],
                recv_sem=recv_sems.at[d],
                device_id=(src,),
                device_id_type=pl.DeviceIdType.MESH,
            )
            recv.wait_recv()
            out_ref[pl.ds(src * m_per, m_per), :] = (
                rbuf[pl.ds(src * m_per, m_per), :].astype(jnp.float32)
            )

        send_rdmas = []
        for d in range(1, N_DEV + 1):
            slot = (d - 1) % 3
            tgt = lax.rem(my + d, N_DEV)
            with jax.named_scope(f"dmawait#d={d}"):
                if d + 2 <= N_DEV:
                    nxt = w_dma((d + 1) % 3, lax.rem(my + d + 2, N_DEV))
                    nxt.start()
                    dmas.append(nxt)
                dmas[d - 1].wait()

            with jax.named_scope(f"dot#d={d}"):
                w8 = wbuf[slot].astype(jnp.float8_e5m2)
                chunk = lax.dot_general(
                    x8_ref[...], w8, (((1,), (0,)), ((), ())),
                    preferred_element_type=jnp.float32,
                ) * scale

                if d == N_DEV:
                    out_ref[pl.ds(my * m_per, m_per), :] = chunk
                else:
                    comm_ref[d - 1, :, :] = chunk.astype(jnp.bfloat16)
            if d < N_DEV:
                with jax.named_scope(f"send#d={d}"):
                    if d == 1:
                        pl.semaphore_wait(barrier_sem, N_DEV - 1)
                    rdma = pltpu.make_async_remote_copy(
                        src_ref=comm_ref.at[d - 1],
                        dst_ref=rbuf.at[pl.ds(my * m_per, m_per), :],
                        send_sem=send_sems.at[d],
                        recv_sem=recv_sems.at[d],
                        device_id=(tgt,),
                        device_id_type=pl.DeviceIdType.MESH,
                    )
                    rdma.start()
                    send_rdmas.append(rdma)

        for d in range(1, N_DEV):
            with jax.named_scope(f"recv#d={d}"):
                wait_recv_block(d)

        with jax.named_scope("waitsend"):
            for rdma in send_rdmas:
                rdma.wait_send()

    return pl.pallas_call(
        body,
        out_shape=jax.ShapeDtypeStruct((m_tot, n_per), jnp.float32),
        in_specs=[
            pl.BlockSpec(memory_space=pl.ANY),
            pl.BlockSpec(memory_space=pl.ANY),
            pl.BlockSpec(memory_space=pltpu.VMEM),
            pl.BlockSpec(memory_space=pltpu.VMEM),
        ],
        out_specs=pl.BlockSpec(memory_space=pltpu.VMEM),
        scratch_shapes=[
            pltpu.VMEM((m_per, k), jnp.float32),
            pltpu.VMEM((m_per, k), jnp.float8_e4m3fn),
            pltpu.VMEM((3, k, n_per), jnp.float32),
            pltpu.VMEM((N_DEV - 1, m_per, n_per), jnp.bfloat16),
            pltpu.VMEM((m_tot, n_per), jnp.bfloat16),
            pltpu.SemaphoreType.DMA,
            pltpu.SemaphoreType.DMA((3,)),
            pltpu.SemaphoreType.DMA((N_DEV,)),
            pltpu.SemaphoreType.DMA((N_DEV,)),
        ],
        compiler_params=pltpu.CompilerParams(
            vmem_limit_bytes=120 * 1024 * 1024,
            collective_id=0,
        ),
    )(x, w_mat, scale_x, scale_w)


# device time: 32819 ns/iter; 1.2205x vs baseline; 1.0016x over previous
import jax
import jax.numpy as jnp
from jax import lax
from jax.experimental import pallas as pl
from jax.experimental.pallas import tpu as pltpu

N_DEV = 8


def kernel(x, w_mat, scale_x, scale_w):
    m_per, k = x.shape
    n = w_mat.shape[1]
    n_per = n // N_DEV
    m_tot = m_per * N_DEV

    def body(x_hbm, w_hbm, sx_ref, sw_ref, out_ref,
             xbuf, x8_ref, wbuf, comm_ref, rbuf,
             xload_sem, load_sems, send_sems, recv_sems):
        my = lax.axis_index("i")
        scale = sx_ref[0] * sw_ref[0]

        barrier_sem = pltpu.get_barrier_semaphore()
        for e in range(1, N_DEV):
            pl.semaphore_signal(
                barrier_sem, inc=1,
                device_id=(lax.rem(my + e, N_DEV),),
                device_id_type=pl.DeviceIdType.MESH,
            )

        def w_dma(slot, tgt):
            return pltpu.make_async_copy(
                w_hbm.at[:, pl.ds(tgt * n_per, n_per)],
                wbuf.at[slot],
                load_sems.at[slot],
            )

        x_dma = pltpu.make_async_copy(x_hbm, xbuf, xload_sem)
        x_dma.start()
        dmas = [w_dma(0, lax.rem(my + 1, N_DEV)),
                w_dma(1, lax.rem(my + 2, N_DEV))]
        dmas[0].start()
        dmas[1].start()
        x_dma.wait()
        x8_ref[...] = xbuf[...].astype(jnp.float8_e4m3fn)

        def wait_recv_block(d):
            src = lax.rem(my - d + N_DEV, N_DEV)
            recv = pltpu.make_async_remote_copy(
                src_ref=comm_ref.at[0],
                dst_ref=rbuf.at[pl.ds(src * m_per, m_per), :],
                send_sem=send_sems.at[0],
                recv_sem=recv_sems.at[d],
                device_id=(src,),
                device_id_type=pl.DeviceIdType.MESH,
            )
            recv.wait_recv()
            out_ref[pl.ds(src * m_per, m_per), :] = (
                rbuf[pl.ds(src * m_per, m_per), :].astype(jnp.float32)
            )

        send_rdmas = []
        for d in range(1, N_DEV + 1):
            slot = (d - 1) % 3
            tgt = lax.rem(my + d, N_DEV)
            if d + 2 <= N_DEV:
                nxt = w_dma((d + 1) % 3, lax.rem(my + d + 2, N_DEV))
                nxt.start()
                dmas.append(nxt)
            dmas[d - 1].wait()

            w8 = wbuf[slot].astype(jnp.float8_e5m2)
            chunk = lax.dot_general(
                x8_ref[...], w8, (((1,), (0,)), ((), ())),
                preferred_element_type=jnp.float32,
            ) * scale

            if d == N_DEV:
                out_ref[pl.ds(my * m_per, m_per), :] = chunk
            else:
                comm_ref[d - 1, :, :] = chunk.astype(jnp.bfloat16)
                if d == 1:
                    pl.semaphore_wait(barrier_sem, N_DEV - 1)
                rdma = pltpu.make_async_remote_copy(
                    src_ref=comm_ref.at[d - 1],
                    dst_ref=rbuf.at[pl.ds(my * m_per, m_per), :],
                    send_sem=send_sems.at[d],
                    recv_sem=recv_sems.at[d],
                    device_id=(tgt,),
                    device_id_type=pl.DeviceIdType.MESH,
                )
                rdma.start()
                send_rdmas.append(rdma)

        for d in range(1, N_DEV):
            wait_recv_block(d)

        for rdma in send_rdmas:
            rdma.wait_send()

    return pl.pallas_call(
        body,
        out_shape=jax.ShapeDtypeStruct((m_tot, n_per), jnp.float32),
        in_specs=[
            pl.BlockSpec(memory_space=pl.ANY),
            pl.BlockSpec(memory_space=pl.ANY),
            pl.BlockSpec(memory_space=pltpu.VMEM),
            pl.BlockSpec(memory_space=pltpu.VMEM),
        ],
        out_specs=pl.BlockSpec(memory_space=pltpu.VMEM),
        scratch_shapes=[
            pltpu.VMEM((m_per, k), jnp.float32),
            pltpu.VMEM((m_per, k), jnp.float8_e4m3fn),
            pltpu.VMEM((3, k, n_per), jnp.float32),
            pltpu.VMEM((N_DEV - 1, m_per, n_per), jnp.bfloat16),
            pltpu.VMEM((m_tot, n_per), jnp.bfloat16),
            pltpu.SemaphoreType.DMA,
            pltpu.SemaphoreType.DMA((3,)),
            pltpu.SemaphoreType.DMA((N_DEV,)),
            pltpu.SemaphoreType.DMA((N_DEV,)),
        ],
        compiler_params=pltpu.CompilerParams(
            vmem_limit_bytes=120 * 1024 * 1024,
            collective_id=0,
        ),
    )(x, w_mat, scale_x, scale_w)
